# baseline (speedup 1.0000x reference)
import functools
import math

import jax
import jax.numpy as jnp
from jax.experimental import pallas as pl
from jax.experimental.pallas import tpu as pltpu
from jax.experimental.pallas import tpu_sc as plsc

B = 2
N = 10000
NP = 10240
E = 160000
D = 64
H = 4
DH = 16
MEM = 61
MP = 64
TILE_E = 2000
TILE_N = 1024
NL = 6
M = B * N


def _mlp(p, x):
    h = jax.nn.relu(x @ p["w1"] + p["b1"])
    return jax.nn.relu(h @ p["w2"] + p["b2"])


def _pos_encode(coords, cols):
    parts = [coords]
    for c in cols:
        pair = coords[:, c:c + 2]
        parts.append(jnp.concatenate(
            [jnp.sin(math.pi * pair[:, :1]), jnp.cos(math.pi * pair[:, 1:2])],
            axis=1))
    return jnp.concatenate(parts, axis=1)


def _att_consts(objects, p):
    kh = (objects @ p["wk"]).reshape(B, MEM, H, DH)
    vh = (objects @ p["wv"]).reshape(B, MEM, H, DH)
    Kt = jnp.zeros((B, D, H * MP), jnp.float32)
    Vt = jnp.zeros((B, H * MP, D), jnp.float32)
    for h in range(H):
        Kt = Kt.at[:, h * DH:(h + 1) * DH, h * MP:h * MP + MEM].set(
            jnp.transpose(kh[:, :, h, :], (0, 2, 1)) / math.sqrt(DH))
        Vt = Vt.at[:, h * MP:h * MP + MEM, h * DH:(h + 1) * DH].set(
            vh[:, :, h, :])
    bias = jnp.tile(jnp.where(jnp.arange(MP) < MEM, 0.0, -1e30), H)[None, :]
    return Kt, Vt, bias


def _lnk(x, ln):
    m = jnp.mean(x, axis=-1, keepdims=True)
    xm = x - m
    v = jnp.mean(xm * xm, axis=-1, keepdims=True)
    return xm / jnp.sqrt(v + 1e-5) * ln[0:1, :] + ln[1:2, :]


def _block_body(x, pos, wq, Kt, vbias, Vt, wo, ln1, fw1, fb1, fw2, fb2, ln2):
    q = x + pos
    s = jnp.dot(jnp.dot(q, wq, preferred_element_type=jnp.float32), Kt,
                preferred_element_type=jnp.float32) + vbias
    att_parts = []
    for h in range(H):
        sh = s[:, h * MP:(h + 1) * MP]
        mx = jnp.max(sh, axis=1, keepdims=True)
        e = jnp.exp(sh - mx)
        att_parts.append(e / jnp.sum(e, axis=1, keepdims=True))
    att = jnp.concatenate(att_parts, axis=1)
    o = jnp.dot(jnp.dot(att, Vt, preferred_element_type=jnp.float32), wo,
                preferred_element_type=jnp.float32)
    x = _lnk(x + o, ln1)
    f = jnp.dot(jax.nn.relu(jnp.dot(x, fw1, preferred_element_type=jnp.float32)
                            + fb1), fw2,
                preferred_element_type=jnp.float32) + fb2
    return _lnk(x + f, ln2)


def _pipeline_kernel(want_ale, n_tiles,
                     enc, pos1, pos2, w1, b1, w2, b2, blk1, blk2, Cref,
                     out, ale_out, easum):
    h0 = jax.nn.relu(jnp.dot(enc[...], w1[...],
                             preferred_element_type=jnp.float32) + b1[...])
    x = jax.nn.relu(jnp.dot(h0, w2[...],
                            preferred_element_type=jnp.float32) + b2[...])
    for pos, blk in ((pos1, blk1), (pos2, blk2)):
        (wq, Kt, vbias, Vt, wo, ln1, fw1, fb1, fw2, fb2, ln2) = blk
        x = _block_body(x, pos[...], wq[...], Kt[0], vbias[...], Vt[0],
                        wo[...], ln1[...], fw1[...], fb1[...], fw2[...],
                        fb2[...], ln2[...])
    out[0] = x
    if want_ale:
        ale_out[0] = jnp.dot(x, Cref[...], preferred_element_type=jnp.float32)
        b_idx = pl.program_id(0)
        t_idx = pl.program_id(1)

        @pl.when(jnp.logical_and(b_idx == 0, t_idx == 0))
        def _():
            easum[...] = jnp.zeros_like(easum)

        easum[0:1, :] += jnp.sum(x, axis=0, keepdims=True)


def _blk_params(p, Kt, Vt, bias):
    return (p["wq"], Kt, bias, Vt, p["wo"],
            jnp.stack([p["ln1_g"], p["ln1_b"]]),
            p["ffn_w1"], p["ffn_b1"][None, :], p["ffn_w2"],
            p["ffn_b2"][None, :],
            jnp.stack([p["ln2_g"], p["ln2_b"]]))


def _run_pipeline(enc, mlp_p, att_ps, objects, C, tile, total, want_ale):
    n_tiles = total // tile
    fin = mlp_p["w1"].shape[0]
    w1 = jnp.zeros((8, 2 * D), jnp.float32).at[:fin].set(mlp_p["w1"])
    encp = jnp.zeros((total, 8), jnp.float32).at[:enc.shape[0], :fin].set(enc)

    blks = []
    poss = []
    for p in att_ps:
        Kt, Vt, bias = _att_consts(objects, p)
        blks.append(_blk_params(p, Kt, Vt, bias))
        pos = p["pos"]
        if pos.shape[0] < total:
            pos = jnp.zeros((total, D), jnp.float32).at[:pos.shape[0]].set(pos)
        poss.append(pos)

    row_spec = lambda w: pl.BlockSpec((tile, w), lambda b, t: (t, 0))
    full = lambda a: pl.BlockSpec(a.shape, lambda b, t: (0,) * a.ndim)
    batched = lambda a: pl.BlockSpec((1,) + a.shape[1:],
                                     lambda b, t: (b,) + (0,) * (a.ndim - 1))

    def blk_specs(blk):
        return tuple(batched(a) if a.ndim == 3 else full(a) for a in blk)

    in_specs = [row_spec(8), row_spec(D), row_spec(D),
                full(w1), full(mlp_p["b1"][None, :]), full(mlp_p["w2"]),
                full(mlp_p["b2"][None, :]),
                blk_specs(blks[0]), blk_specs(blks[1]), full(C)]
    out_shapes = [jax.ShapeDtypeStruct((B, total, D), jnp.float32),
                  jax.ShapeDtypeStruct((B, total, 8), jnp.float32),
                  jax.ShapeDtypeStruct((8, D), jnp.float32)]
    out_specs = [pl.BlockSpec((1, tile, D), lambda b, t: (b, t, 0)),
                 pl.BlockSpec((1, tile, 8), lambda b, t: (b, t, 0)),
                 pl.BlockSpec((8, D), lambda b, t: (0, 0))]

    outs = pl.pallas_call(
        functools.partial(_pipeline_kernel, want_ale, n_tiles),
        grid=(B, n_tiles),
        in_specs=in_specs,
        out_specs=out_specs,
        out_shape=out_shapes,
    )(encp, poss[0], poss[1], w1, mlp_p["b1"][None, :], mlp_p["w2"],
      mlp_p["b2"][None, :], blks[0], blks[1], C)
    return outs


def _pack_h(h, al, rows):
    h80 = jnp.concatenate(
        [h, al[:, 0:1], jnp.zeros((rows, 15), jnp.float32)], axis=1)
    alw = jnp.concatenate(
        [al[:, 1:2], al[:, 0:1] + al[:, 1:2],
         jnp.zeros((rows, 14), jnp.float32)], axis=1)
    return h80, alw


def _gat_first_kernel(x, W, A, h_out, alw_out):
    h = jnp.dot(x[...], W[...], preferred_element_type=jnp.float32)
    al = jnp.dot(h, A[...], preferred_element_type=jnp.float32)
    h80, alw = _pack_h(h, al, x.shape[0])
    h_out[...] = h80
    alw_out[...] = alw


def _gat_first(x, W, A):
    nt = M // _TM
    row = lambda w: pl.BlockSpec((_TM, w), lambda t: (t, 0))
    full = lambda a: pl.BlockSpec(a.shape, lambda t: (0,) * a.ndim)
    return pl.pallas_call(
        _gat_first_kernel,
        grid=(nt,),
        in_specs=[row(D), full(W), full(A)],
        out_specs=[row(_W), row(16)],
        out_shape=[jax.ShapeDtypeStruct((M, _W), jnp.float32),
                   jax.ShapeDtypeStruct((M, 16), jnp.float32)],
    )(x, W, A)


_TM = 4000


def _gat_stats_kernel(numer, h_prev, bias, out_ref, stats):
    num80 = numer[0] + numer[1]
    seg = num80[:, :D] + h_prev[:, :D]
    den = num80[:, D:D + 1] + (1.0 + 1e-16)
    out = seg / den + bias[...]
    out_ref[...] = out

    @pl.when(pl.program_id(0) == 0)
    def _():
        stats[...] = jnp.zeros_like(stats)

    stats[0:1, :] += jnp.sum(out, axis=0, keepdims=True)
    stats[1:2, :] += jnp.sum(out * out, axis=0, keepdims=True)


def _gat_apply_kernel(want_next, out, stats, bn, W, A, h_out, alw_out=None):
    mn = stats[0:1, :] * (1.0 / M)
    vr = stats[1:2, :] * (1.0 / M) - mn * mn
    x = (out[...] - mn) / jnp.sqrt(vr + 1e-5) * bn[0:1, :] + bn[1:2, :]
    if want_next:
        h = jnp.dot(x, W[...], preferred_element_type=jnp.float32)
        al = jnp.dot(h, A[...], preferred_element_type=jnp.float32)
        h80, alw = _pack_h(h, al, x.shape[0])
        h_out[...] = h80
        alw_out[...] = alw
    else:
        h_out[...] = x


def _gat_epilogue(numer, h_prev, bias, bn, W, A, want_next):
    nt = M // _TM
    row = lambda w: pl.BlockSpec((_TM, w), lambda t: (t, 0))
    full = lambda a: pl.BlockSpec(a.shape, lambda t: (0,) * a.ndim)
    out, stats = pl.pallas_call(
        _gat_stats_kernel,
        grid=(nt,),
        in_specs=[pl.BlockSpec((2, _TM, _W), lambda t: (0, t, 0)),
                  row(_W), full(bias)],
        out_specs=[row(D), pl.BlockSpec((2, D), lambda t: (0, 0))],
        out_shape=[jax.ShapeDtypeStruct((M, D), jnp.float32),
                   jax.ShapeDtypeStruct((2, D), jnp.float32)],
    )(numer, h_prev, bias)
    if want_next:
        out_shape = [jax.ShapeDtypeStruct((M, _W), jnp.float32),
                     jax.ShapeDtypeStruct((M, 16), jnp.float32)]
        out_specs = [row(_W), row(16)]
    else:
        out_shape = [jax.ShapeDtypeStruct((M, D), jnp.float32)]
        out_specs = [row(D)]
    res = pl.pallas_call(
        functools.partial(_gat_apply_kernel, want_next),
        grid=(nt,),
        in_specs=[row(D), full(stats), full(bn), full(W), full(A)],
        out_specs=out_specs,
        out_shape=out_shape,
    )(out, stats, bn, W, A)
    return res if want_next else res[0]


_NC = 2
_NS = 16
_EC = (B * E) // (_NC * _NS)
_ROWS = M // _NS
_K = 5
_CH = 400
_W = 80


_NSC = _CH // (16 * _K)


def _sc_gat_body(h_hbm, alw_hbm, src_hbm, dst2_hbm, ale_hbm, aleloop_hbm,
                 zrows_hbm, numer_out,
                 src_v, dst_v, ale_v, dstrow_v, rowbuf, ex_v, aleloop_v,
                 numer_s, gsem, ssem):
    cid = jax.lax.axis_index("c")
    sid = jax.lax.axis_index("s")
    wid = cid * _NS + sid
    base = wid * _EC

    pltpu.sync_copy(aleloop_hbm, aleloop_v)
    pltpu.sync_copy(zrows_hbm, numer_s.at[pl.ds(sid * _ROWS, _ROWS)])
    plsc.subcore_barrier()
    alv = aleloop_v[...]
    iota = jax.lax.iota(jnp.int32, 16)
    unit = (iota == 0).astype(jnp.float32)

    def chunk(ci):
        cbase = base + ci * _CH
        pltpu.sync_copy(src_hbm.at[pl.ds(cbase, _CH)], src_v)
        pltpu.sync_copy(dst2_hbm.at[pl.ds(cbase // 80, _NSC)], dst_v)
        pltpu.sync_copy(ale_hbm.at[pl.ds(cbase, _CH)], ale_v)
        for k in range(_NSC):
            cpk = pltpu.async_copy(alw_hbm.at[dst_v.at[k]],
                                   dstrow_v.at[pl.ds(k * 80, 80)], gsem)
        for k in range(_NSC):
            cpk.wait()

        def superchunk(t):
            for j in range(_K):
                g = t * _K + j
                sidx = src_v[pl.ds(g * 16, 16)]
                cp = pltpu.async_copy(
                    h_hbm.at[sidx], rowbuf.at[pl.ds(j * 16, 16)], gsem)
            for j in range(_K):
                cp.wait()
            for j in range(_K):
                g = t * _K + j
                a_s = plsc.load_gather(
                    rowbuf, [iota + j * 16, jnp.full((16,), D, jnp.int32)])
                drow = iota + g * 16
                a_d = plsc.load_gather(
                    dstrow_v, [drow, jnp.zeros((16,), jnp.int32)])
                sal = plsc.load_gather(
                    dstrow_v, [drow, jnp.ones((16,), jnp.int32)])
                z = a_s + a_d + ale_v[pl.ds(g * 16, 16)]
                e = jnp.where(z > 0, z, 0.2 * z)
                zl = sal + alv
                c = jnp.where(zl > 0, zl, 0.2 * zl)
                ex = jnp.exp(e - c)
                ex_v[pl.ds(16, 16)] = ex
                for r in range(16):
                    row = jnp.full((16,), j * 16 + r, jnp.int32)
                    w = plsc.load_gather(
                        ex_v, [jnp.full((16,), 16 + r, jnp.int32)])
                    for cc in range(D // 16):
                        cols = cc * 16 + iota
                        v = plsc.load_gather(rowbuf, [row, cols])
                        plsc.store_scatter(rowbuf, [row, cols], v * w)
                    plsc.store_scatter(rowbuf, [row, D + iota], w * unit)
            for j in range(_K):
                didx = dst_v[t, pl.ds(j * 16, 16)]
                cp2 = pltpu.async_copy(
                    rowbuf.at[pl.ds(j * 16, 16)],
                    numer_s.at[didx], ssem, add=True)
            for j in range(_K):
                cp2.wait()

        pl.loop(0, _NSC)(superchunk)

    pl.loop(0, _EC // _CH)(chunk)
    plsc.subcore_barrier()
    pltpu.sync_copy(numer_s.at[pl.ds(sid * _ROWS, _ROWS)],
                    numer_out.at[cid, pl.ds(sid * _ROWS, _ROWS)])


def _sc_gat(h80, alw, src, dst2, ale, aleloop, zrows):
    mesh = plsc.VectorSubcoreMesh(core_axis_name="c", subcore_axis_name="s",
                                  num_cores=_NC, num_subcores=_NS)
    return pl.kernel(
        _sc_gat_body, mesh=mesh,
        compiler_params=pltpu.CompilerParams(use_tc_tiling_on_sc=False,
                                             needs_layout_passes=False),
        out_type=jax.ShapeDtypeStruct((_NC, M, _W), jnp.float32),
        scratch_types=[
            pltpu.VMEM((_CH,), jnp.int32),
            pltpu.VMEM((_NSC, 80), jnp.int32),
            pltpu.VMEM((_CH,), jnp.float32),
            pltpu.VMEM((_CH, 16), jnp.float32),
            pltpu.VMEM((16 * _K, _W), jnp.float32),
            pltpu.VMEM((32,), jnp.float32),
            pltpu.VMEM((16,), jnp.float32),
            pltpu.VMEM_SHARED((M, _W), jnp.float32),
            pltpu.SemaphoreType.DMA,
            pltpu.SemaphoreType.DMA,
        ])(h80, alw, src, dst2, ale, aleloop, zrows)


def kernel(agvs, stat, nodes, paths, params):
    f32 = jnp.float32
    agvs = agvs.astype(f32)
    stat = stat.astype(f32)
    nodes = nodes.astype(f32)

    obs_main = agvs[:, :1]
    obs_agvs = agvs[:, 1:]
    coords = obs_main[:, :, 8:16].reshape(-1, 4, 2)
    d2 = ((coords[:, :, None, :] - nodes[None, None, :, :]) ** 2).sum(-1)
    in_reach = jnp.argmin(d2, axis=-1)
    objects = jnp.concatenate([
        _mlp(params["embedd_main"], obs_main),
        _mlp(params["embedd_agv"], obs_agvs),
        _mlp(params["embedd_station"], stat)], axis=1)

    C = jnp.stack([p["We"] @ p["a_e"] for p in params["convs"]], axis=1)
    C = jnp.concatenate([C, jnp.zeros((D, 8 - NL), f32)], axis=1)

    nodes_enc = _pos_encode(nodes, [0])
    edge_coords = jnp.concatenate([nodes[paths[0]], nodes[paths[1]]], axis=1)
    edges_enc = _pos_encode(edge_coords, [0, 2])

    nodes_emb, _, _ = _run_pipeline(
        nodes_enc, params["embedd_node"], params["node_att"], objects, C,
        TILE_N, NP, want_ale=False)
    nodes_emb = nodes_emb[:, :N]
    edges_emb, ALE, easum = _run_pipeline(
        edges_enc, params["embedd_edge"], params["edge_att"], objects, C,
        TILE_E, E, want_ale=True)
    ale_loop = easum[0] / (B * E) @ C

    offsets = jnp.arange(B, dtype=paths.dtype) * N
    src = (paths[0][None, :] + offsets[:, None]).reshape(-1)
    dst = (paths[1][None, :] + offsets[:, None]).reshape(-1)
    ALE2 = ALE.reshape(B * E, 8)

    src = src.astype(jnp.int32)
    dst2 = dst.astype(jnp.int32).reshape(-1, 80)
    zrows = jnp.zeros((_ROWS, _W), f32)

    x0 = nodes_emb.reshape(M, D)
    convs = params["convs"]

    def amat(p):
        a = jnp.zeros((D, 8), f32)
        return a.at[:, 0].set(p["a_src"]).at[:, 1].set(p["a_dst"])

    h80, alw = _gat_first(x0, convs[0]["W"], amat(convs[0]))
    for li in range(NL):
        p = convs[li]
        aleloop = jnp.full((16,), ale_loop[li], f32)
        numer = _sc_gat(h80, alw, src, dst2, ALE2[:, li], aleloop, zrows)
        bias = p["bias"][None, :]
        bn = jnp.stack([p["bn_g"], p["bn_b"]])
        if li < NL - 1:
            pn = convs[li + 1]
            h80, alw = _gat_epilogue(numer, h80, bias, bn, pn["W"],
                                     amat(pn), True)
        else:
            x = _gat_epilogue(numer, h80, bias, bn,
                              jnp.zeros((1, 1), f32), jnp.zeros((1, 1), f32),
                              False)

    conv = x.reshape(B, N, D)
    idx = jnp.broadcast_to(in_reach[:, :, None], (B, 4, D))
    return jnp.take_along_axis(conv, idx, axis=1).reshape(B, -1)


# speedup vs baseline: 1.0781x; 1.0781x over previous
import functools
import math

import jax
import jax.numpy as jnp
from jax.experimental import pallas as pl
from jax.experimental.pallas import tpu as pltpu
from jax.experimental.pallas import tpu_sc as plsc

B = 2
N = 10000
NP = 10240
E = 160000
D = 64
H = 4
DH = 16
MEM = 61
MP = 64
TILE_E = 2000
TILE_N = 1024
NL = 6
M = B * N


def _mlp(p, x):
    h = jax.nn.relu(x @ p["w1"] + p["b1"])
    return jax.nn.relu(h @ p["w2"] + p["b2"])


def _pos_encode(coords, cols):
    parts = [coords]
    for c in cols:
        pair = coords[:, c:c + 2]
        parts.append(jnp.concatenate(
            [jnp.sin(math.pi * pair[:, :1]), jnp.cos(math.pi * pair[:, 1:2])],
            axis=1))
    return jnp.concatenate(parts, axis=1)


def _att_consts(objects, p):
    kh = (objects @ p["wk"]).reshape(B, MEM, H, DH)
    vh = (objects @ p["wv"]).reshape(B, MEM, H, DH)
    Kt = jnp.zeros((B, D, H * MP), jnp.float32)
    Vt = jnp.zeros((B, H * MP, D), jnp.float32)
    for h in range(H):
        Kt = Kt.at[:, h * DH:(h + 1) * DH, h * MP:h * MP + MEM].set(
            jnp.transpose(kh[:, :, h, :], (0, 2, 1)) / math.sqrt(DH))
        Vt = Vt.at[:, h * MP:h * MP + MEM, h * DH:(h + 1) * DH].set(
            vh[:, :, h, :])
    bias = jnp.tile(jnp.where(jnp.arange(MP) < MEM, 0.0, -1e30), H)[None, :]
    return Kt, Vt, bias


def _lnk(x, ln):
    m = jnp.mean(x, axis=-1, keepdims=True)
    xm = x - m
    v = jnp.mean(xm * xm, axis=-1, keepdims=True)
    return xm / jnp.sqrt(v + 1e-5) * ln[0:1, :] + ln[1:2, :]


def _block_body(x, pos, wq, Kt, vbias, Vt, wo, ln1, fw1, fb1, fw2, fb2, ln2):
    q = x + pos
    s = jnp.dot(jnp.dot(q, wq, preferred_element_type=jnp.float32), Kt,
                preferred_element_type=jnp.float32) + vbias
    att_parts = []
    for h in range(H):
        sh = s[:, h * MP:(h + 1) * MP]
        mx = jnp.max(sh, axis=1, keepdims=True)
        e = jnp.exp(sh - mx)
        att_parts.append(e / jnp.sum(e, axis=1, keepdims=True))
    att = jnp.concatenate(att_parts, axis=1)
    o = jnp.dot(jnp.dot(att, Vt, preferred_element_type=jnp.float32), wo,
                preferred_element_type=jnp.float32)
    x = _lnk(x + o, ln1)
    f = jnp.dot(jax.nn.relu(jnp.dot(x, fw1, preferred_element_type=jnp.float32)
                            + fb1), fw2,
                preferred_element_type=jnp.float32) + fb2
    return _lnk(x + f, ln2)


def _pipeline_kernel(want_ale, n_tiles,
                     enc, pos1, pos2, w1, b1, w2, b2, blk1, blk2, Cref,
                     out, ale_out, easum):
    h0 = jax.nn.relu(jnp.dot(enc[...], w1[...],
                             preferred_element_type=jnp.float32) + b1[...])
    x = jax.nn.relu(jnp.dot(h0, w2[...],
                            preferred_element_type=jnp.float32) + b2[...])
    for pos, blk in ((pos1, blk1), (pos2, blk2)):
        (wq, Kt, vbias, Vt, wo, ln1, fw1, fb1, fw2, fb2, ln2) = blk
        x = _block_body(x, pos[...], wq[...], Kt[0], vbias[...], Vt[0],
                        wo[...], ln1[...], fw1[...], fb1[...], fw2[...],
                        fb2[...], ln2[...])
    out[0] = x
    if want_ale:
        ale_out[0] = jnp.dot(x, Cref[...], preferred_element_type=jnp.float32)
        b_idx = pl.program_id(0)
        t_idx = pl.program_id(1)

        @pl.when(jnp.logical_and(b_idx == 0, t_idx == 0))
        def _():
            easum[...] = jnp.zeros_like(easum)

        easum[0:1, :] += jnp.sum(x, axis=0, keepdims=True)


def _blk_params(p, Kt, Vt, bias):
    return (p["wq"], Kt, bias, Vt, p["wo"],
            jnp.stack([p["ln1_g"], p["ln1_b"]]),
            p["ffn_w1"], p["ffn_b1"][None, :], p["ffn_w2"],
            p["ffn_b2"][None, :],
            jnp.stack([p["ln2_g"], p["ln2_b"]]))


def _run_pipeline(enc, mlp_p, att_ps, objects, C, tile, total, want_ale):
    n_tiles = total // tile
    fin = mlp_p["w1"].shape[0]
    w1 = jnp.zeros((8, 2 * D), jnp.float32).at[:fin].set(mlp_p["w1"])
    encp = jnp.zeros((total, 8), jnp.float32).at[:enc.shape[0], :fin].set(enc)

    blks = []
    poss = []
    for p in att_ps:
        Kt, Vt, bias = _att_consts(objects, p)
        blks.append(_blk_params(p, Kt, Vt, bias))
        pos = p["pos"]
        if pos.shape[0] < total:
            pos = jnp.zeros((total, D), jnp.float32).at[:pos.shape[0]].set(pos)
        poss.append(pos)

    row_spec = lambda w: pl.BlockSpec((tile, w), lambda b, t: (t, 0))
    full = lambda a: pl.BlockSpec(a.shape, lambda b, t: (0,) * a.ndim)
    batched = lambda a: pl.BlockSpec((1,) + a.shape[1:],
                                     lambda b, t: (b,) + (0,) * (a.ndim - 1))

    def blk_specs(blk):
        return tuple(batched(a) if a.ndim == 3 else full(a) for a in blk)

    in_specs = [row_spec(8), row_spec(D), row_spec(D),
                full(w1), full(mlp_p["b1"][None, :]), full(mlp_p["w2"]),
                full(mlp_p["b2"][None, :]),
                blk_specs(blks[0]), blk_specs(blks[1]), full(C)]
    out_shapes = [jax.ShapeDtypeStruct((B, total, D), jnp.float32),
                  jax.ShapeDtypeStruct((B, total, 8), jnp.float32),
                  jax.ShapeDtypeStruct((8, D), jnp.float32)]
    out_specs = [pl.BlockSpec((1, tile, D), lambda b, t: (b, t, 0)),
                 pl.BlockSpec((1, tile, 8), lambda b, t: (b, t, 0)),
                 pl.BlockSpec((8, D), lambda b, t: (0, 0))]

    outs = pl.pallas_call(
        functools.partial(_pipeline_kernel, want_ale, n_tiles),
        grid=(B, n_tiles),
        in_specs=in_specs,
        out_specs=out_specs,
        out_shape=out_shapes,
    )(encp, poss[0], poss[1], w1, mlp_p["b1"][None, :], mlp_p["w2"],
      mlp_p["b2"][None, :], blks[0], blks[1], C)
    return outs


def _pack_h(h, al, rows):
    h80 = jnp.concatenate(
        [h, al[:, 0:1], jnp.zeros((rows, 15), jnp.float32)], axis=1)
    alw = jnp.concatenate(
        [al[:, 1:2], al[:, 0:1] + al[:, 1:2],
         jnp.zeros((rows, 14), jnp.float32)], axis=1)
    return h80, alw


def _gat_first_kernel(x, W, A, h_out, alw_out):
    h = jnp.dot(x[...], W[...], preferred_element_type=jnp.float32)
    al = jnp.dot(h, A[...], preferred_element_type=jnp.float32)
    h80, alw = _pack_h(h, al, x.shape[0])
    h_out[...] = h80
    alw_out[...] = alw


def _gat_first(x, W, A):
    nt = M // _TM
    row = lambda w: pl.BlockSpec((_TM, w), lambda t: (t, 0))
    full = lambda a: pl.BlockSpec(a.shape, lambda t: (0,) * a.ndim)
    return pl.pallas_call(
        _gat_first_kernel,
        grid=(nt,),
        in_specs=[row(D), full(W), full(A)],
        out_specs=[row(_W), row(16)],
        out_shape=[jax.ShapeDtypeStruct((M, _W), jnp.float32),
                   jax.ShapeDtypeStruct((M, 16), jnp.float32)],
    )(x, W, A)


_TM = 4000


def _gat_stats_kernel(numer, h_prev, bias, out_ref, stats):
    num80 = numer[0] + numer[1]
    seg = num80[:, :D] + h_prev[:, :D]
    den = num80[:, D:D + 1] + (1.0 + 1e-16)
    out = seg / den + bias[...]
    out_ref[...] = out

    @pl.when(pl.program_id(0) == 0)
    def _():
        stats[...] = jnp.zeros_like(stats)

    stats[0:1, :] += jnp.sum(out, axis=0, keepdims=True)
    stats[1:2, :] += jnp.sum(out * out, axis=0, keepdims=True)


def _gat_apply_kernel(want_next, out, stats, bn, W, A, h_out, alw_out=None):
    mn = stats[0:1, :] * (1.0 / M)
    vr = stats[1:2, :] * (1.0 / M) - mn * mn
    x = (out[...] - mn) / jnp.sqrt(vr + 1e-5) * bn[0:1, :] + bn[1:2, :]
    if want_next:
        h = jnp.dot(x, W[...], preferred_element_type=jnp.float32)
        al = jnp.dot(h, A[...], preferred_element_type=jnp.float32)
        h80, alw = _pack_h(h, al, x.shape[0])
        h_out[...] = h80
        alw_out[...] = alw
    else:
        h_out[...] = x


def _gat_epilogue(numer, h_prev, bias, bn, W, A, want_next):
    nt = M // _TM
    row = lambda w: pl.BlockSpec((_TM, w), lambda t: (t, 0))
    full = lambda a: pl.BlockSpec(a.shape, lambda t: (0,) * a.ndim)
    out, stats = pl.pallas_call(
        _gat_stats_kernel,
        grid=(nt,),
        in_specs=[pl.BlockSpec((2, _TM, _W), lambda t: (0, t, 0)),
                  row(_W), full(bias)],
        out_specs=[row(D), pl.BlockSpec((2, D), lambda t: (0, 0))],
        out_shape=[jax.ShapeDtypeStruct((M, D), jnp.float32),
                   jax.ShapeDtypeStruct((2, D), jnp.float32)],
    )(numer, h_prev, bias)
    if want_next:
        out_shape = [jax.ShapeDtypeStruct((M, _W), jnp.float32),
                     jax.ShapeDtypeStruct((M, 16), jnp.float32)]
        out_specs = [row(_W), row(16)]
    else:
        out_shape = [jax.ShapeDtypeStruct((M, D), jnp.float32)]
        out_specs = [row(D)]
    res = pl.pallas_call(
        functools.partial(_gat_apply_kernel, want_next),
        grid=(nt,),
        in_specs=[row(D), full(stats), full(bn), full(W), full(A)],
        out_specs=out_specs,
        out_shape=out_shape,
    )(out, stats, bn, W, A)
    return res if want_next else res[0]


_NC = 2
_NS = 16
_EC = (B * E) // (_NC * _NS)
_ROWS = M // _NS
_K = 5
_CH = 400
_W = 80


_NSC = _CH // (16 * _K)


def _sc_gat_body(li, h_hbm, alw_hbm, src2_hbm, dst2_hbm, ale2_hbm,
                 aleloop_hbm, zrows_hbm, numer_out,
                 src_v, dst_v, ale2_v, dstrow_v, rowbuf, ex_v, aleloop_v,
                 numer_s, gsem, dsem, ssem):
    cid = jax.lax.axis_index("c")
    sid = jax.lax.axis_index("s")
    wid = cid * _NS + sid
    base = wid * _EC

    pltpu.sync_copy(aleloop_hbm, aleloop_v)
    pltpu.sync_copy(zrows_hbm, numer_s.at[pl.ds(sid * _ROWS, _ROWS)])
    plsc.subcore_barrier()
    alv = aleloop_v[...]
    iota = jax.lax.iota(jnp.int32, 16)
    unit = (iota == 0).astype(jnp.float32)
    licol = jnp.full((16,), li, jnp.int32)

    def chunk(ci):
        cbase = base + ci * _CH
        rb = cbase // 80
        c1 = pltpu.async_copy(src2_hbm.at[pl.ds(rb, _NSC)], src_v, dsem)
        c2 = pltpu.async_copy(dst2_hbm.at[pl.ds(rb, _NSC)], dst_v, dsem)
        c3 = pltpu.async_copy(ale2_hbm.at[pl.ds(cbase, _CH)], ale2_v, dsem)
        c1.wait()
        c2.wait()
        c3.wait()

        def superchunk(t):
            cpg = pltpu.async_copy(h_hbm.at[src_v.at[t]], rowbuf, gsem)
            cpd = pltpu.async_copy(alw_hbm.at[dst_v.at[t]], dstrow_v, dsem)
            cpg.wait()
            cpd.wait()
            for g in range(_K):
                rows16 = iota + g * 16
                a_s = plsc.load_gather(
                    rowbuf, [rows16, jnp.full((16,), D, jnp.int32)])
                a_d = plsc.load_gather(
                    dstrow_v, [rows16, jnp.zeros((16,), jnp.int32)])
                sal = plsc.load_gather(
                    dstrow_v, [rows16, jnp.ones((16,), jnp.int32)])
                ale = plsc.load_gather(
                    ale2_v, [iota + (t * 80 + g * 16), licol])
                z = a_s + a_d + ale
                e = jnp.where(z > 0, z, 0.2 * z)
                zl = sal + alv
                c = jnp.where(zl > 0, zl, 0.2 * zl)
                ex = jnp.exp(e - c)
                ex_v[pl.ds(16, 16)] = ex
                for r in range(16):
                    row = jnp.full((16,), g * 16 + r, jnp.int32)
                    w = plsc.load_gather(
                        ex_v, [jnp.full((16,), 16 + r, jnp.int32)])
                    for cc in range(D // 16):
                        cols = cc * 16 + iota
                        v = plsc.load_gather(rowbuf, [row, cols])
                        plsc.store_scatter(rowbuf, [row, cols], v * w)
                    plsc.store_scatter(rowbuf, [row, D + iota], w * unit)
            cps = pltpu.async_copy(rowbuf, numer_s.at[dst_v.at[t]],
                                   ssem, add=True)
            cps.wait()

        pl.loop(0, _NSC)(superchunk)

    pl.loop(0, _EC // _CH)(chunk)
    plsc.subcore_barrier()
    pltpu.sync_copy(numer_s.at[pl.ds(sid * _ROWS, _ROWS)],
                    numer_out.at[cid, pl.ds(sid * _ROWS, _ROWS)])


def _sc_gat(li, h80, alw, src2, dst2, ale2, aleloop, zrows):
    mesh = plsc.VectorSubcoreMesh(core_axis_name="c", subcore_axis_name="s",
                                  num_cores=_NC, num_subcores=_NS)
    return pl.kernel(
        functools.partial(_sc_gat_body, li), mesh=mesh,
        compiler_params=pltpu.CompilerParams(use_tc_tiling_on_sc=False,
                                             needs_layout_passes=False),
        out_type=jax.ShapeDtypeStruct((_NC, M, _W), jnp.float32),
        scratch_types=[
            pltpu.VMEM((_NSC, 80), jnp.int32),
            pltpu.VMEM((_NSC, 80), jnp.int32),
            pltpu.VMEM((_CH, 8), jnp.float32),
            pltpu.VMEM((80, 16), jnp.float32),
            pltpu.VMEM((80, _W), jnp.float32),
            pltpu.VMEM((32,), jnp.float32),
            pltpu.VMEM((16,), jnp.float32),
            pltpu.VMEM_SHARED((M, _W), jnp.float32),
            pltpu.SemaphoreType.DMA,
            pltpu.SemaphoreType.DMA,
            pltpu.SemaphoreType.DMA,
        ])(h80, alw, src2, dst2, ale2, aleloop, zrows)


def kernel(agvs, stat, nodes, paths, params):
    f32 = jnp.float32
    agvs = agvs.astype(f32)
    stat = stat.astype(f32)
    nodes = nodes.astype(f32)

    obs_main = agvs[:, :1]
    obs_agvs = agvs[:, 1:]
    coords = obs_main[:, :, 8:16].reshape(-1, 4, 2)
    d2 = ((coords[:, :, None, :] - nodes[None, None, :, :]) ** 2).sum(-1)
    in_reach = jnp.argmin(d2, axis=-1)
    objects = jnp.concatenate([
        _mlp(params["embedd_main"], obs_main),
        _mlp(params["embedd_agv"], obs_agvs),
        _mlp(params["embedd_station"], stat)], axis=1)

    C = jnp.stack([p["We"] @ p["a_e"] for p in params["convs"]], axis=1)
    C = jnp.concatenate([C, jnp.zeros((D, 8 - NL), f32)], axis=1)

    nodes_enc = _pos_encode(nodes, [0])
    edge_coords = jnp.concatenate([nodes[paths[0]], nodes[paths[1]]], axis=1)
    edges_enc = _pos_encode(edge_coords, [0, 2])

    nodes_emb, _, _ = _run_pipeline(
        nodes_enc, params["embedd_node"], params["node_att"], objects, C,
        TILE_N, NP, want_ale=False)
    nodes_emb = nodes_emb[:, :N]
    edges_emb, ALE, easum = _run_pipeline(
        edges_enc, params["embedd_edge"], params["edge_att"], objects, C,
        TILE_E, E, want_ale=True)
    ale_loop = easum[0] / (B * E) @ C

    offsets = jnp.arange(B, dtype=paths.dtype) * N
    src = (paths[0][None, :] + offsets[:, None]).reshape(-1)
    dst = (paths[1][None, :] + offsets[:, None]).reshape(-1)
    ALE2 = ALE.reshape(B * E, 8)

    src2 = src.astype(jnp.int32).reshape(-1, 80)
    dst2 = dst.astype(jnp.int32).reshape(-1, 80)
    zrows = jnp.zeros((_ROWS, _W), f32)

    x0 = nodes_emb.reshape(M, D)
    convs = params["convs"]

    def amat(p):
        a = jnp.zeros((D, 8), f32)
        return a.at[:, 0].set(p["a_src"]).at[:, 1].set(p["a_dst"])

    h80, alw = _gat_first(x0, convs[0]["W"], amat(convs[0]))
    for li in range(NL):
        p = convs[li]
        aleloop = jnp.full((16,), ale_loop[li], f32)
        numer = _sc_gat(li, h80, alw, src2, dst2, ALE2, aleloop, zrows)
        bias = p["bias"][None, :]
        bn = jnp.stack([p["bn_g"], p["bn_b"]])
        if li < NL - 1:
            pn = convs[li + 1]
            h80, alw = _gat_epilogue(numer, h80, bias, bn, pn["W"],
                                     amat(pn), True)
        else:
            x = _gat_epilogue(numer, h80, bias, bn,
                              jnp.zeros((1, 1), f32), jnp.zeros((1, 1), f32),
                              False)

    conv = x.reshape(B, N, D)
    idx = jnp.broadcast_to(in_reach[:, :, None], (B, 4, D))
    return jnp.take_along_axis(conv, idx, axis=1).reshape(B, -1)


# speedup vs baseline: 1.2609x; 1.1695x over previous
import functools
import math

import jax
import jax.numpy as jnp
from jax.experimental import pallas as pl
from jax.experimental.pallas import tpu as pltpu
from jax.experimental.pallas import tpu_sc as plsc

B = 2
N = 10000
NP = 10240
E = 160000
D = 64
H = 4
DH = 16
MEM = 61
MP = 64
TILE_E = 2000
TILE_N = 1024
NL = 6
M = B * N


def _mlp(p, x):
    h = jax.nn.relu(x @ p["w1"] + p["b1"])
    return jax.nn.relu(h @ p["w2"] + p["b2"])


def _pos_encode(coords, cols):
    parts = [coords]
    for c in cols:
        pair = coords[:, c:c + 2]
        parts.append(jnp.concatenate(
            [jnp.sin(math.pi * pair[:, :1]), jnp.cos(math.pi * pair[:, 1:2])],
            axis=1))
    return jnp.concatenate(parts, axis=1)


def _att_consts(objects, p):
    kh = (objects @ p["wk"]).reshape(B, MEM, H, DH)
    vh = (objects @ p["wv"]).reshape(B, MEM, H, DH)
    Kt = jnp.zeros((B, D, H * MP), jnp.float32)
    Vt = jnp.zeros((B, H * MP, D), jnp.float32)
    for h in range(H):
        Kt = Kt.at[:, h * DH:(h + 1) * DH, h * MP:h * MP + MEM].set(
            jnp.transpose(kh[:, :, h, :], (0, 2, 1)) / math.sqrt(DH))
        Vt = Vt.at[:, h * MP:h * MP + MEM, h * DH:(h + 1) * DH].set(
            vh[:, :, h, :])
    bias = jnp.tile(jnp.where(jnp.arange(MP) < MEM, 0.0, -1e30), H)[None, :]
    return Kt, Vt, bias


def _lnk(x, ln):
    m = jnp.mean(x, axis=-1, keepdims=True)
    xm = x - m
    v = jnp.mean(xm * xm, axis=-1, keepdims=True)
    return xm / jnp.sqrt(v + 1e-5) * ln[0:1, :] + ln[1:2, :]


def _block_body(x, pos, wq, Kt, vbias, Vt, wo, ln1, fw1, fb1, fw2, fb2, ln2):
    q = x + pos
    s = jnp.dot(jnp.dot(q, wq, preferred_element_type=jnp.float32), Kt,
                preferred_element_type=jnp.float32) + vbias
    att_parts = []
    for h in range(H):
        sh = s[:, h * MP:(h + 1) * MP]
        mx = jnp.max(sh, axis=1, keepdims=True)
        e = jnp.exp(sh - mx)
        att_parts.append(e / jnp.sum(e, axis=1, keepdims=True))
    att = jnp.concatenate(att_parts, axis=1)
    o = jnp.dot(jnp.dot(att, Vt, preferred_element_type=jnp.float32), wo,
                preferred_element_type=jnp.float32)
    x = _lnk(x + o, ln1)
    f = jnp.dot(jax.nn.relu(jnp.dot(x, fw1, preferred_element_type=jnp.float32)
                            + fb1), fw2,
                preferred_element_type=jnp.float32) + fb2
    return _lnk(x + f, ln2)


def _pipeline_kernel(want_ale, n_tiles,
                     enc, g1, pos1, pos2, w1, b1, w2, b2, blk1, blk2, Cref,
                     out, ale_out, easum):
    if want_ale:
        c = jnp.concatenate([enc[...], g1[...]], axis=1)
        encv = jnp.concatenate(
            [c,
             jnp.sin(math.pi * c[:, 0:1]), jnp.cos(math.pi * c[:, 1:2]),
             jnp.sin(math.pi * c[:, 2:3]), jnp.cos(math.pi * c[:, 3:4])],
            axis=1)
    else:
        encv = enc[...]
    h0 = jax.nn.relu(jnp.dot(encv, w1[...],
                             preferred_element_type=jnp.float32) + b1[...])
    x = jax.nn.relu(jnp.dot(h0, w2[...],
                            preferred_element_type=jnp.float32) + b2[...])
    for pos, blk in ((pos1, blk1), (pos2, blk2)):
        (wq, Kt, vbias, Vt, wo, ln1, fw1, fb1, fw2, fb2, ln2) = blk
        x = _block_body(x, pos[...], wq[...], Kt[0], vbias[...], Vt[0],
                        wo[...], ln1[...], fw1[...], fb1[...], fw2[...],
                        fb2[...], ln2[...])
    out[0] = x
    if want_ale:
        ale_out[0] = jnp.dot(x, Cref[...], preferred_element_type=jnp.float32)
        b_idx = pl.program_id(0)
        t_idx = pl.program_id(1)

        @pl.when(jnp.logical_and(b_idx == 0, t_idx == 0))
        def _():
            easum[...] = jnp.zeros_like(easum)

        easum[0:1, :] += jnp.sum(x, axis=0, keepdims=True)


def _blk_params(p, Kt, Vt, bias):
    return (p["wq"], Kt, bias, Vt, p["wo"],
            jnp.stack([p["ln1_g"], p["ln1_b"]]),
            p["ffn_w1"], p["ffn_b1"][None, :], p["ffn_w2"],
            p["ffn_b2"][None, :],
            jnp.stack([p["ln2_g"], p["ln2_b"]]))


def _run_pipeline(enc, g1, mlp_p, att_ps, objects, C, tile, total, want_ale):
    n_tiles = total // tile
    fin = mlp_p["w1"].shape[0]
    w1 = jnp.zeros((8, 2 * D), jnp.float32).at[:fin].set(mlp_p["w1"])
    if want_ale:
        encp = enc
    else:
        encp = jnp.zeros((total, 8), jnp.float32).at[:enc.shape[0], :fin].set(
            enc)

    blks = []
    poss = []
    for p in att_ps:
        Kt, Vt, bias = _att_consts(objects, p)
        blks.append(_blk_params(p, Kt, Vt, bias))
        pos = p["pos"]
        if pos.shape[0] < total:
            pos = jnp.zeros((total, D), jnp.float32).at[:pos.shape[0]].set(pos)
        poss.append(pos)

    row_spec = lambda w: pl.BlockSpec((tile, w), lambda b, t: (t, 0))
    full = lambda a: pl.BlockSpec(a.shape, lambda b, t: (0,) * a.ndim)
    batched = lambda a: pl.BlockSpec((1,) + a.shape[1:],
                                     lambda b, t: (b,) + (0,) * (a.ndim - 1))

    def blk_specs(blk):
        return tuple(batched(a) if a.ndim == 3 else full(a) for a in blk)

    in_specs = [row_spec(encp.shape[1]), row_spec(g1.shape[1]),
                row_spec(D), row_spec(D),
                full(w1), full(mlp_p["b1"][None, :]), full(mlp_p["w2"]),
                full(mlp_p["b2"][None, :]),
                blk_specs(blks[0]), blk_specs(blks[1]), full(C)]
    out_shapes = [jax.ShapeDtypeStruct((B, total, D), jnp.float32),
                  jax.ShapeDtypeStruct((B, total, 8), jnp.float32),
                  jax.ShapeDtypeStruct((8, D), jnp.float32)]
    out_specs = [pl.BlockSpec((1, tile, D), lambda b, t: (b, t, 0)),
                 pl.BlockSpec((1, tile, 8), lambda b, t: (b, t, 0)),
                 pl.BlockSpec((8, D), lambda b, t: (0, 0))]

    outs = pl.pallas_call(
        functools.partial(_pipeline_kernel, want_ale, n_tiles),
        grid=(B, n_tiles),
        in_specs=in_specs,
        out_specs=out_specs,
        out_shape=out_shapes,
    )(encp, g1, poss[0], poss[1], w1, mlp_p["b1"][None, :], mlp_p["w2"],
      mlp_p["b2"][None, :], blks[0], blks[1], C)
    return outs


def _pack_h(h, al, rows):
    h80 = jnp.concatenate(
        [h, al[:, 0:1], jnp.zeros((rows, 15), jnp.float32)], axis=1)
    alw = jnp.concatenate(
        [al[:, 1:2], al[:, 0:1] + al[:, 1:2],
         jnp.zeros((rows, 14), jnp.float32)], axis=1)
    return h80, alw


def _gat_first_kernel(x, W, A, h_out, alw_out):
    h = jnp.dot(x[...], W[...], preferred_element_type=jnp.float32)
    al = jnp.dot(h, A[...], preferred_element_type=jnp.float32)
    h80, alw = _pack_h(h, al, x.shape[0])
    h_out[...] = h80
    alw_out[...] = alw


def _gat_first(x, W, A):
    nt = M // _TM
    row = lambda w: pl.BlockSpec((_TM, w), lambda t: (t, 0))
    full = lambda a: pl.BlockSpec(a.shape, lambda t: (0,) * a.ndim)
    return pl.pallas_call(
        _gat_first_kernel,
        grid=(nt,),
        in_specs=[row(D), full(W), full(A)],
        out_specs=[row(_W), row(16)],
        out_shape=[jax.ShapeDtypeStruct((M, _W), jnp.float32),
                   jax.ShapeDtypeStruct((M, 16), jnp.float32)],
    )(x, W, A)


_TM = 4000


def _gat_stats_kernel(numer, h_prev, bias, out_ref, stats):
    num80 = numer[0] + numer[1]
    seg = num80[:, :D] + h_prev[:, :D]
    den = num80[:, D:D + 1] + (1.0 + 1e-16)
    out = seg / den + bias[...]
    out_ref[...] = out

    @pl.when(pl.program_id(0) == 0)
    def _():
        stats[...] = jnp.zeros_like(stats)

    stats[0:1, :] += jnp.sum(out, axis=0, keepdims=True)
    stats[1:2, :] += jnp.sum(out * out, axis=0, keepdims=True)


def _gat_apply_kernel(want_next, out, stats, bn, W, A, h_out, alw_out=None):
    mn = stats[0:1, :] * (1.0 / M)
    vr = stats[1:2, :] * (1.0 / M) - mn * mn
    x = (out[...] - mn) / jnp.sqrt(vr + 1e-5) * bn[0:1, :] + bn[1:2, :]
    if want_next:
        h = jnp.dot(x, W[...], preferred_element_type=jnp.float32)
        al = jnp.dot(h, A[...], preferred_element_type=jnp.float32)
        h80, alw = _pack_h(h, al, x.shape[0])
        h_out[...] = h80
        alw_out[...] = alw
    else:
        h_out[...] = x


def _gat_epilogue(numer, h_prev, bias, bn, W, A, want_next):
    nt = M // _TM
    row = lambda w: pl.BlockSpec((_TM, w), lambda t: (t, 0))
    full = lambda a: pl.BlockSpec(a.shape, lambda t: (0,) * a.ndim)
    out, stats = pl.pallas_call(
        _gat_stats_kernel,
        grid=(nt,),
        in_specs=[pl.BlockSpec((2, _TM, _W), lambda t: (0, t, 0)),
                  row(_W), full(bias)],
        out_specs=[row(D), pl.BlockSpec((2, D), lambda t: (0, 0))],
        out_shape=[jax.ShapeDtypeStruct((M, D), jnp.float32),
                   jax.ShapeDtypeStruct((2, D), jnp.float32)],
    )(numer, h_prev, bias)
    if want_next:
        out_shape = [jax.ShapeDtypeStruct((M, _W), jnp.float32),
                     jax.ShapeDtypeStruct((M, 16), jnp.float32)]
        out_specs = [row(_W), row(16)]
    else:
        out_shape = [jax.ShapeDtypeStruct((M, D), jnp.float32)]
        out_specs = [row(D)]
    res = pl.pallas_call(
        functools.partial(_gat_apply_kernel, want_next),
        grid=(nt,),
        in_specs=[row(D), full(stats), full(bn), full(W), full(A)],
        out_specs=out_specs,
        out_shape=out_shape,
    )(out, stats, bn, W, A)
    return res if want_next else res[0]


_NC = 2
_NS = 16
_EC = (B * E) // (_NC * _NS)
_ROWS = M // _NS
_K = 5
_CH = 400
_W = 80


_NSC = _CH // (16 * _K)


def _sc_gat_body(li, h_hbm, alw_hbm, src2_hbm, dst2_hbm, ale2_hbm,
                 aleloop_hbm, zrows_hbm, numer_out,
                 src_v, dst_v, ale2_v, dstrow_v, rowbuf, ex_v, aleloop_v,
                 numer_s, gsem, dsem, ssem):
    cid = jax.lax.axis_index("c")
    sid = jax.lax.axis_index("s")
    wid = cid * _NS + sid
    base = wid * _EC

    pltpu.sync_copy(aleloop_hbm, aleloop_v)
    pltpu.sync_copy(zrows_hbm, numer_s.at[pl.ds(sid * _ROWS, _ROWS)])
    plsc.subcore_barrier()
    alv = aleloop_v[...]
    iota = jax.lax.iota(jnp.int32, 16)
    unit = (iota == 0).astype(jnp.float32)
    licol = jnp.full((16,), li, jnp.int32)

    def chunk(ci):
        cbase = base + ci * _CH
        rb = cbase // 80
        c1 = pltpu.async_copy(src2_hbm.at[pl.ds(rb, _NSC)], src_v, dsem)
        c2 = pltpu.async_copy(dst2_hbm.at[pl.ds(rb, _NSC)], dst_v, dsem)
        c3 = pltpu.async_copy(ale2_hbm.at[pl.ds(cbase, _CH)], ale2_v, dsem)
        c1.wait()
        c2.wait()
        c3.wait()

        def superchunk(t):
            cpg = pltpu.async_copy(h_hbm.at[src_v.at[t]], rowbuf, gsem)
            cpd = pltpu.async_copy(alw_hbm.at[dst_v.at[t]], dstrow_v, dsem)
            cpg.wait()
            cpd.wait()
            for g in range(_K):
                rows16 = iota + g * 16
                a_s = plsc.load_gather(
                    rowbuf, [rows16, jnp.full((16,), D, jnp.int32)])
                a_d = plsc.load_gather(
                    dstrow_v, [rows16, jnp.zeros((16,), jnp.int32)])
                sal = plsc.load_gather(
                    dstrow_v, [rows16, jnp.ones((16,), jnp.int32)])
                ale = plsc.load_gather(
                    ale2_v, [iota + (t * 80 + g * 16), licol])
                z = a_s + a_d + ale
                e = jnp.where(z > 0, z, 0.2 * z)
                zl = sal + alv
                c = jnp.where(zl > 0, zl, 0.2 * zl)
                ex = jnp.exp(e - c)
                ex_v[pl.ds(16, 16)] = ex
                for r in range(16):
                    row = jnp.full((16,), g * 16 + r, jnp.int32)
                    w = plsc.load_gather(
                        ex_v, [jnp.full((16,), 16 + r, jnp.int32)])
                    for cc in range(D // 16):
                        cols = cc * 16 + iota
                        v = plsc.load_gather(rowbuf, [row, cols])
                        plsc.store_scatter(rowbuf, [row, cols], v * w)
                    plsc.store_scatter(rowbuf, [row, D + iota], w * unit)
            cps = pltpu.async_copy(rowbuf, numer_s.at[dst_v.at[t]],
                                   ssem, add=True)
            cps.wait()

        pl.loop(0, _NSC)(superchunk)

    pl.loop(0, _EC // _CH)(chunk)
    plsc.subcore_barrier()
    pltpu.sync_copy(numer_s.at[pl.ds(sid * _ROWS, _ROWS)],
                    numer_out.at[cid, pl.ds(sid * _ROWS, _ROWS)])


def _sc_gat(li, h80, alw, src2, dst2, ale2, aleloop, zrows):
    mesh = plsc.VectorSubcoreMesh(core_axis_name="c", subcore_axis_name="s",
                                  num_cores=_NC, num_subcores=_NS)
    return pl.kernel(
        functools.partial(_sc_gat_body, li), mesh=mesh,
        compiler_params=pltpu.CompilerParams(use_tc_tiling_on_sc=False,
                                             needs_layout_passes=False),
        out_type=jax.ShapeDtypeStruct((_NC, M, _W), jnp.float32),
        scratch_types=[
            pltpu.VMEM((_NSC, 80), jnp.int32),
            pltpu.VMEM((_NSC, 80), jnp.int32),
            pltpu.VMEM((_CH, 8), jnp.float32),
            pltpu.VMEM((80, 16), jnp.float32),
            pltpu.VMEM((80, _W), jnp.float32),
            pltpu.VMEM((32,), jnp.float32),
            pltpu.VMEM((16,), jnp.float32),
            pltpu.VMEM_SHARED((M, _W), jnp.float32),
            pltpu.SemaphoreType.DMA,
            pltpu.SemaphoreType.DMA,
            pltpu.SemaphoreType.DMA,
        ])(h80, alw, src2, dst2, ale2, aleloop, zrows)


def kernel(agvs, stat, nodes, paths, params):
    f32 = jnp.float32
    agvs = agvs.astype(f32)
    stat = stat.astype(f32)
    nodes = nodes.astype(f32)

    obs_main = agvs[:, :1]
    obs_agvs = agvs[:, 1:]
    coords = obs_main[:, :, 8:16].reshape(-1, 4, 2)
    d2 = ((coords[:, :, None, :] - nodes[None, None, :, :]) ** 2).sum(-1)
    in_reach = jnp.argmin(d2, axis=-1)
    objects = jnp.concatenate([
        _mlp(params["embedd_main"], obs_main),
        _mlp(params["embedd_agv"], obs_agvs),
        _mlp(params["embedd_station"], stat)], axis=1)

    C = jnp.stack([p["We"] @ p["a_e"] for p in params["convs"]], axis=1)
    C = jnp.concatenate([C, jnp.zeros((D, 8 - NL), f32)], axis=1)

    nodes_enc = _pos_encode(nodes, [0])
    ecoord0 = nodes[paths[0]]
    ecoord1 = nodes[paths[1]]

    nodes_emb, _, _ = _run_pipeline(
        nodes_enc, jnp.zeros((NP, 2), f32), params["embedd_node"],
        params["node_att"], objects, C, TILE_N, NP, want_ale=False)
    nodes_emb = nodes_emb[:, :N]
    edges_emb, ALE, easum = _run_pipeline(
        ecoord0, ecoord1, params["embedd_edge"], params["edge_att"],
        objects, C, TILE_E, E, want_ale=True)
    ale_loop = easum[0] / (B * E) @ C

    offsets = jnp.arange(B, dtype=paths.dtype) * N
    src = (paths[0][None, :] + offsets[:, None]).reshape(-1)
    dst = (paths[1][None, :] + offsets[:, None]).reshape(-1)
    ALE2 = ALE.reshape(B * E, 8)

    src2 = src.astype(jnp.int32).reshape(-1, 80)
    dst2 = dst.astype(jnp.int32).reshape(-1, 80)
    zrows = jnp.zeros((_ROWS, _W), f32)

    x0 = nodes_emb.reshape(M, D)
    convs = params["convs"]

    def amat(p):
        a = jnp.zeros((D, 8), f32)
        return a.at[:, 0].set(p["a_src"]).at[:, 1].set(p["a_dst"])

    h80, alw = _gat_first(x0, convs[0]["W"], amat(convs[0]))
    for li in range(NL):
        p = convs[li]
        aleloop = jnp.full((16,), ale_loop[li], f32)
        numer = _sc_gat(li, h80, alw, src2, dst2, ALE2, aleloop, zrows)
        bias = p["bias"][None, :]
        bn = jnp.stack([p["bn_g"], p["bn_b"]])
        if li < NL - 1:
            pn = convs[li + 1]
            h80, alw = _gat_epilogue(numer, h80, bias, bn, pn["W"],
                                     amat(pn), True)
        else:
            x = _gat_epilogue(numer, h80, bias, bn,
                              jnp.zeros((1, 1), f32), jnp.zeros((1, 1), f32),
                              False)

    conv = x.reshape(B, N, D)
    idx = jnp.broadcast_to(in_reach[:, :, None], (B, 4, D))
    return jnp.take_along_axis(conv, idx, axis=1).reshape(B, -1)
